# Initial kernel scaffold; baseline (speedup 1.0000x reference)
import jax
import jax.numpy as jnp
from jax.experimental import pallas as pl


def kernel(x, edge_index, gcn0_w, gcn0_b, gcn1_w, gcn1_b, proj_w, proj_b, conv1_w, conv1_b, conv2_w, conv2_b, lin_w, lin_b):
    raise NotImplementedError("write your pallas kernel here")


# speedup vs baseline: 1.0016x; 1.0016x over previous
import jax
import jax.numpy as jnp
from jax.experimental import pallas as pl

N = 10000
D = 128
K = 5


def _gcn_agg(x, src, dst, W, b):
    xw = x @ W
    deg = jnp.zeros((N,), dtype=x.dtype).at[dst].add(1.0)
    dinv = jnp.where(deg > 0, deg ** -0.5, 0.0)
    norm = dinv[src] * dinv[dst]
    msg = xw[src] * norm[:, None]
    out = jnp.zeros_like(xw).at[dst].add(msg)
    return out + b


def _conv1d_shift(x, w, b):
    pad = (K - 1) // 2
    xp = jnp.pad(x, ((pad, pad), (0, 0)))
    out = jnp.zeros((N, w.shape[0]), x.dtype)
    for k in range(K):
        out = out + xp[k:k + N] @ w[:, :, k].T
    return out + b[None, :]


def _final_linear_kernel(h1_ref, h2_ref, wa_ref, wb_ref, b_ref, o_ref):
    o_ref[...] = (h1_ref[...] @ wa_ref[...] + h2_ref[...] @ wb_ref[...]
                  + b_ref[...])


def kernel(x, edge_index, gcn0_w, gcn0_b, gcn1_w, gcn1_b, proj_w, proj_b,
           conv1_w, conv1_b, conv2_w, conv2_b, lin_w, lin_b):
    loops = jnp.arange(N, dtype=edge_index.dtype)
    src = jnp.concatenate([edge_index[0], loops])
    dst = jnp.concatenate([edge_index[1], loops])
    h = jax.nn.relu(_gcn_agg(x, src, dst, gcn0_w, gcn0_b))
    h1 = _gcn_agg(h, src, dst, gcn1_w, gcn1_b)
    g1 = (h1 @ proj_w.T + proj_b)[:, 0]
    order = jnp.argsort(g1)
    inv = jnp.argsort(order)
    sorted_x = (g1[order])[:, None] * h1[order]
    sx = jax.nn.relu(_conv1d_shift(sorted_x, conv1_w, conv1_b))
    sx = _conv1d_shift(sx, conv2_w, conv2_b)
    h2 = sx[inv]
    out = pl.pallas_call(
        _final_linear_kernel,
        out_shape=jax.ShapeDtypeStruct((N, D), jnp.float32),
    )(h1, h2, lin_w[:, :D].T, lin_w[:, D:].T, jnp.broadcast_to(lin_b, (N, D)))
    return out


# speedup vs baseline: 21.9910x; 21.9552x over previous
import functools

import jax
import jax.numpy as jnp
from jax import lax
from jax.experimental import pallas as pl
from jax.experimental.pallas import tpu as pltpu
from jax.experimental.pallas import tpu_sc as plsc

N = 10000
E = 320000
D = 128
K = 5
NPAD = 10240
EP = 327680
ER = EP // 128
EPAD = EP - E
HALF = 64
EPT = EP // 16
RPT = EPT // 128
HRPT = ER // 32
GB = 256
NB = EPT // GB
NRT = NPAD // 16

_MESH = plsc.VectorSubcoreMesh(core_axis_name="c", subcore_axis_name="s",
                               num_cores=2, num_subcores=16)


@functools.partial(
    pl.kernel,
    out_type=jax.ShapeDtypeStruct((2 * NPAD,), jnp.float32),
    mesh=_MESH,
    scratch_types=[
        pltpu.VMEM((HRPT, 128), jnp.int32),
        pltpu.VMEM((128,), jnp.float32),
        pltpu.VMEM_SHARED((NPAD,), jnp.float32),
    ],
)
def _sc_hist(dst2d, init_half, out, idx_v, ones_v, hist_sp):
    c = lax.axis_index("c")
    s = lax.axis_index("s")

    @pl.when(s == 0)
    def _():
        pltpu.sync_copy(init_half, hist_sp)

    for i in range(8):
        ones_v[pl.ds(i * 16, 16)] = jnp.ones((16,), jnp.float32)

    pltpu.sync_copy(dst2d.at[pl.ds((c * 16 + s) * HRPT, HRPT)], idx_v)
    plsc.subcore_barrier()

    def body(j, carry):
        pltpu.sync_copy(ones_v, hist_sp.at[idx_v.at[j]], add=True)
        return carry

    lax.fori_loop(0, HRPT, body, 0)
    plsc.subcore_barrier()

    @pl.when(s == 0)
    def _():
        pltpu.sync_copy(hist_sp, out.at[pl.ds(c * NPAD, NPAD)])


@functools.partial(
    pl.kernel,
    out_type=jax.ShapeDtypeStruct((2 * NPAD, HALF), jnp.float32),
    mesh=_MESH,
    scratch_types=[
        pltpu.VMEM((EPT,), jnp.int32),
        pltpu.VMEM((RPT, 128), jnp.int32),
        pltpu.VMEM((GB, HALF), jnp.float32),
        pltpu.VMEM((GB, HALF), jnp.float32),
        pltpu.VMEM_SHARED((NPAD, HALF), jnp.float32),
        pltpu.SemaphoreType.DMA,
        pltpu.SemaphoreType.DMA,
        pltpu.SemaphoreType.DMA,
    ],
    compiler_params=pltpu.CompilerParams(use_tc_tiling_on_sc=False),
)
def _sc_rowscat(y2, src2, dst2d, out, src_v, dst_v, rows_a, rows_b, acc_sp,
                sem_a, sem_b, sem_s):
    c = lax.axis_index("c")
    s = lax.axis_index("s")

    pltpu.sync_copy(y2.at[pl.ds(c * NPAD + s * NRT, NRT)],
                    acc_sp.at[pl.ds(s * NRT, NRT)])
    pltpu.sync_copy(src2.at[pl.ds(c * EP + s * EPT, EPT)], src_v)
    pltpu.sync_copy(dst2d.at[pl.ds(s * RPT, RPT)], dst_v)
    plsc.subcore_barrier()

    def _issue(b, buf, sem):
        pltpu.async_copy(y2.at[src_v.at[pl.ds(b * GB, GB)]], buf, sem)

    def _wait(b, buf, sem):
        pltpu.make_async_copy(y2.at[src_v.at[pl.ds(b * GB, GB)]], buf,
                              sem).wait()

    def _scat(b, buf):
        for j in range(GB // 128):
            pltpu.async_copy(buf.at[pl.ds(j * 128, 128)],
                             acc_sp.at[dst_v.at[b * (GB // 128) + j]],
                             sem_s, add=True)
        for j in range(GB // 128):
            pltpu.make_async_copy(buf.at[pl.ds(j * 128, 128)],
                                  acc_sp.at[dst_v.at[b * (GB // 128) + j]],
                                  sem_s).wait()

    _issue(0, rows_a, sem_a)

    def body(b2, carry):
        b0 = 2 * b2
        _issue(b0 + 1, rows_b, sem_b)
        _wait(b0, rows_a, sem_a)
        _scat(b0, rows_a)

        @pl.when(b0 + 2 < NB)
        def _():
            _issue(b0 + 2, rows_a, sem_a)

        _wait(b0 + 1, rows_b, sem_b)
        _scat(b0 + 1, rows_b)
        return carry

    lax.fori_loop(0, NB // 2, body, 0)
    plsc.subcore_barrier()
    pltpu.sync_copy(acc_sp.at[pl.ds(s * NRT, NRT)],
                    out.at[pl.ds(c * NPAD + s * NRT, NRT)])


@functools.partial(
    pl.kernel,
    out_type=jax.ShapeDtypeStruct((NPAD, D), jnp.float32),
    mesh=_MESH,
    scratch_types=[
        pltpu.VMEM((8, 128), jnp.int32),
        pltpu.VMEM((8, 128), jnp.int32),
        pltpu.VMEM((NRT,), jnp.int32),
        pltpu.VMEM((NRT, D), jnp.float32),
        pltpu.VMEM_SHARED((NPAD,), jnp.int32),
        pltpu.SemaphoreType.DMA,
    ],
)
def _sc_sortperm(rank2d, iota2d, sxu, out, rank_v, vals_v, idx_v, rows_v,
                 order_sp, sem):
    c = lax.axis_index("c")
    s = lax.axis_index("s")

    @pl.when(c == 0)
    def _():
        @pl.when(s < 10)
        def _():
            pltpu.sync_copy(rank2d.at[pl.ds(s * 8, 8)], rank_v)
            pltpu.sync_copy(iota2d.at[pl.ds(s * 8, 8)], vals_v)
            for j in range(8):
                pltpu.sync_copy(vals_v.at[j], order_sp.at[rank_v.at[j]])

        plsc.subcore_barrier()
        base = s * NRT
        pltpu.sync_copy(order_sp.at[pl.ds(base, NRT)], idx_v)
        pltpu.async_copy(sxu.at[idx_v], rows_v, sem).wait()
        pltpu.sync_copy(rows_v, out.at[pl.ds(base, NRT)])


@functools.partial(
    pl.kernel,
    out_type=jax.ShapeDtypeStruct((NPAD, D), jnp.float32),
    mesh=_MESH,
    scratch_types=[
        pltpu.VMEM((NRT,), jnp.int32),
        pltpu.VMEM((NRT, D), jnp.float32),
        pltpu.SemaphoreType.DMA,
    ],
)
def _sc_pgather(table, idx, out, idx_v, rows_v, sem):
    c = lax.axis_index("c")
    s = lax.axis_index("s")

    @pl.when(c == 0)
    def _():
        base = s * NRT
        pltpu.sync_copy(idx.at[pl.ds(base, NRT)], idx_v)
        pltpu.async_copy(table.at[idx_v], rows_v, sem).wait()
        pltpu.sync_copy(rows_v, out.at[pl.ds(base, NRT)])


def _tc_xw_body(x_ref, w_ref, o_ref):
    o_ref[...] = jnp.dot(x_ref[...], w_ref[...],
                         preferred_element_type=jnp.float32)


def _tc_y0_body(xw_ref, h_ref, y_ref):
    hh = h_ref[...]
    dinv = lax.rsqrt(hh[:, 0] + hh[:, 1])[:, None]
    y = xw_ref[...] * dinv
    y_ref[0] = y[:, :HALF]
    y_ref[1] = y[:, HALF:]


def _tc_mid_body(s_ref, h_ref, b_ref, w_ref, y_ref):
    hh = h_ref[...]
    dinv = lax.rsqrt(hh[:, 0] + hh[:, 1])[:, None]
    sfull = jnp.concatenate([s_ref[0], s_ref[1]], axis=1)
    hrelu = jnp.maximum(dinv * sfull + b_ref[...][None, :], 0.0)
    y = jnp.dot(hrelu, w_ref[...],
                preferred_element_type=jnp.float32) * dinv
    y_ref[0] = y[:, :HALF]
    y_ref[1] = y[:, HALF:]


def _tc_head_body(s_ref, h_ref, b_ref, p_ref, pb_ref,
                  h1_ref, kc_ref, sxu_ref):
    i = pl.program_id(0)
    B = 2048
    hh = h_ref[...]
    dinv = lax.rsqrt(hh[:, 0] + hh[:, 1])[:, None]
    sfull = jnp.concatenate([s_ref[0], s_ref[1]], axis=1)
    h1 = dinv * sfull + b_ref[...][None, :]
    g = jnp.sum(h1 * p_ref[...][None, :], axis=1, keepdims=True) + pb_ref[0, 0]
    rows = i * B + lax.broadcasted_iota(jnp.int32, (B, 1), 0)
    valid = rows < N
    bits = lax.bitcast_convert_type(g, jnp.int32)
    key = bits ^ (lax.shift_right_arithmetic(bits, 31) & jnp.int32(0x7FFFFFFF))
    h1_ref[...] = h1
    kc_ref[...] = jnp.where(valid, key, jnp.int32(0x7FFFFFFF))
    sxu_ref[...] = jnp.where(valid, g * h1, 0.0)


_RBI, _RBJ = 2048, 2048


def _tc_rank_body(kc_ref, kr_ref, o_ref):
    i = pl.program_id(0)
    j = pl.program_id(1)
    BI, BJ = _RBI, _RBJ
    ki = kc_ref[...]
    kj = kr_ref[...]

    @pl.when(j == 0)
    def _():
        o_ref[...] = jnp.zeros((BI, 1), jnp.int32)

    @pl.when(j < i)
    def _():
        o_ref[...] += jnp.sum((kj <= ki).astype(jnp.int32), axis=1,
                              keepdims=True)

    @pl.when(j > i)
    def _():
        o_ref[...] += jnp.sum((kj < ki).astype(jnp.int32), axis=1,
                              keepdims=True)

    @pl.when(j == i)
    def _():
        ii = i * BI + lax.broadcasted_iota(jnp.int32, (BI, 1), 0)
        jj = j * BJ + lax.broadcasted_iota(jnp.int32, (1, BJ), 1)
        cnt = ((kj < ki) | ((kj == ki) & (jj < ii))).astype(jnp.int32)
        o_ref[...] += jnp.sum(cnt, axis=1, keepdims=True)


def _tc_conv_body(prev_ref, cur_ref, next_ref, w1_ref, b1_ref,
                  w2_ref, b2_ref, o_ref):
    i = pl.program_id(0)
    B = 2048
    ext = jnp.concatenate(
        [prev_ref[...][B - 4:], cur_ref[...], next_ref[...][:4]], axis=0)
    r = i * B - 4 + lax.broadcasted_iota(jnp.int32, (B + 8, 1), 0)
    ext = jnp.where((r >= 0) & (r < N), ext, 0.0)
    z = jnp.zeros((B + 4, D), jnp.float32)
    for k in range(K):
        z = z + jnp.dot(ext[k:k + B + 4], w1_ref[k],
                        preferred_element_type=jnp.float32)
    z = jnp.maximum(z + b1_ref[...][None, :], 0.0)
    zr = i * B - 2 + lax.broadcasted_iota(jnp.int32, (B + 4, 1), 0)
    z = jnp.where((zr >= 0) & (zr < N), z, 0.0)
    out = jnp.zeros((B, D), jnp.float32)
    for k in range(K):
        out = out + jnp.dot(z[k:k + B], w2_ref[k],
                            preferred_element_type=jnp.float32)
    o_ref[...] = out + b2_ref[...][None, :]


def _tc_parta_body(h1_ref, la_ref, b_ref, o_ref):
    o_ref[...] = (jnp.dot(h1_ref[...], la_ref[...],
                          preferred_element_type=jnp.float32)
                  + b_ref[...][None, :])


def _tc_final_body(pa_ref, h2_ref, lb_ref, o_ref):
    o_ref[...] = pa_ref[...] + jnp.dot(
        h2_ref[...], lb_ref[...], preferred_element_type=jnp.float32)


def _full(shape):
    zeros = (0,) * len(shape)
    return pl.BlockSpec(shape, lambda *_: zeros)


def _tc_xw(x, w0):
    B = 2048
    return pl.pallas_call(
        _tc_xw_body,
        grid=(NPAD // B,),
        in_specs=[pl.BlockSpec((B, D), lambda i: (i, 0)), _full((D, D))],
        out_specs=pl.BlockSpec((B, D), lambda i: (i, 0)),
        out_shape=jax.ShapeDtypeStruct((NPAD, D), jnp.float32),
    )(x, w0)


def _tc_y0(xw, hists):
    B = 2048
    return pl.pallas_call(
        _tc_y0_body,
        grid=(NPAD // B,),
        in_specs=[pl.BlockSpec((B, D), lambda i: (i, 0)),
                  pl.BlockSpec((B, 2), lambda i: (i, 0))],
        out_specs=pl.BlockSpec((2, B, HALF), lambda i: (0, i, 0)),
        out_shape=jax.ShapeDtypeStruct((2, NPAD, HALF), jnp.float32),
    )(xw, hists)


def _tc_mid(s0, hists, b0, w1):
    B = 2048
    return pl.pallas_call(
        _tc_mid_body,
        grid=(NPAD // B,),
        in_specs=[pl.BlockSpec((2, B, HALF), lambda i: (0, i, 0)),
                  pl.BlockSpec((B, 2), lambda i: (i, 0)),
                  _full((D,)), _full((D, D))],
        out_specs=pl.BlockSpec((2, B, HALF), lambda i: (0, i, 0)),
        out_shape=jax.ShapeDtypeStruct((2, NPAD, HALF), jnp.float32),
    )(s0, hists, b0, w1)


def _tc_head(s1, hists, b1, proj_row, pb):
    B = 2048
    return pl.pallas_call(
        _tc_head_body,
        grid=(NPAD // B,),
        in_specs=[pl.BlockSpec((2, B, HALF), lambda i: (0, i, 0)),
                  pl.BlockSpec((B, 2), lambda i: (i, 0)),
                  _full((D,)), _full((D,)), _full((1, 1))],
        out_specs=[pl.BlockSpec((B, D), lambda i: (i, 0)),
                   pl.BlockSpec((B, 1), lambda i: (i, 0)),
                   pl.BlockSpec((B, D), lambda i: (i, 0))],
        out_shape=[jax.ShapeDtypeStruct((N, D), jnp.float32),
                   jax.ShapeDtypeStruct((NPAD, 1), jnp.int32),
                   jax.ShapeDtypeStruct((NPAD, D), jnp.float32)],
    )(s1, hists, b1, proj_row, pb)


def _tc_rank(keys_col, keys_row):
    BI, BJ = _RBI, _RBJ
    return pl.pallas_call(
        _tc_rank_body,
        grid=(NPAD // BI, NPAD // BJ),
        in_specs=[pl.BlockSpec((BI, 1), lambda i, j: (i, 0)),
                  pl.BlockSpec((1, BJ), lambda i, j: (0, j))],
        out_specs=pl.BlockSpec((BI, 1), lambda i, j: (i, 0)),
        out_shape=jax.ShapeDtypeStruct((NPAD, 1), jnp.int32),
    )(keys_col, keys_row)


def _tc_conv(sx, w1k, b1, w2k, b2):
    B = 2048
    nb = NPAD // B
    return pl.pallas_call(
        _tc_conv_body,
        grid=(nb,),
        in_specs=[
            pl.BlockSpec((B, D), lambda i: (jnp.maximum(i - 1, 0), 0)),
            pl.BlockSpec((B, D), lambda i: (i, 0)),
            pl.BlockSpec((B, D), lambda i: (jnp.minimum(i + 1, nb - 1), 0)),
            _full((K, D, D)), _full((D,)), _full((K, D, D)), _full((D,)),
        ],
        out_specs=pl.BlockSpec((B, D), lambda i: (i, 0)),
        out_shape=jax.ShapeDtypeStruct((NPAD, D), jnp.float32),
    )(sx, sx, sx, w1k, b1, w2k, b2)


def _tc_parta(h1, la, lbias):
    B = 2000
    return pl.pallas_call(
        _tc_parta_body,
        grid=(N // B,),
        in_specs=[pl.BlockSpec((B, D), lambda i: (i, 0)),
                  _full((D, D)), _full((D,))],
        out_specs=pl.BlockSpec((B, D), lambda i: (i, 0)),
        out_shape=jax.ShapeDtypeStruct((N, D), jnp.float32),
    )(h1, la, lbias)


def _tc_final(pa, h2, lb):
    B = 2000
    return pl.pallas_call(
        _tc_final_body,
        grid=(N // B,),
        in_specs=[pl.BlockSpec((B, D), lambda i: (i, 0)),
                  pl.BlockSpec((B, D), lambda i: (i, 0)),
                  _full((D, D))],
        out_specs=pl.BlockSpec((B, D), lambda i: (i, 0)),
        out_shape=jax.ShapeDtypeStruct((N, D), jnp.float32),
    )(pa, h2, lb)


def kernel(x, edge_index, gcn0_w, gcn0_b, gcn1_w, gcn1_b, proj_w, proj_b,
           conv1_w, conv1_b, conv2_w, conv2_b, lin_w, lin_b):
    import numpy as _np
    pad_k = _np.arange(EPAD, dtype=_np.int32) % 64
    src1d = jnp.concatenate([edge_index[0], jnp.asarray(pad_k)])
    src2 = jnp.concatenate([src1d, src1d + NPAD])
    dst2d = jnp.concatenate(
        [edge_index[1], jnp.asarray(N + pad_k)]).reshape(ER, 128)
    init_half = jnp.full((NPAD,), 0.5, jnp.float32)
    iota2d = jnp.asarray(
        _np.arange(NPAD, dtype=_np.int32).reshape(NPAD // 128, 128))
    w1k = jnp.transpose(conv1_w, (2, 1, 0))
    w2k = jnp.transpose(conv2_w, (2, 1, 0))
    la = lin_w[:, :D].T
    lb = lin_w[:, D:].T

    hflat = _sc_hist(dst2d, init_half)
    hists = hflat.reshape(2, NPAD).T
    xw = _tc_xw(x, gcn0_w)
    y0 = _tc_y0(xw, hists)
    s0 = _sc_rowscat(y0.reshape(2 * NPAD, HALF),
                     src2, dst2d).reshape(2, NPAD, HALF)
    y1 = _tc_mid(s0, hists, gcn0_b, gcn1_w)
    s1 = _sc_rowscat(y1.reshape(2 * NPAD, HALF),
                     src2, dst2d).reshape(2, NPAD, HALF)
    h1, keys_col, sxu = _tc_head(s1, hists, gcn1_b,
                                 proj_w[0], proj_b.reshape(1, 1))
    keys_row = keys_col.reshape(1, NPAD)
    rank_col = _tc_rank(keys_col, keys_row)
    rank1d = rank_col.reshape(NPAD)
    rank2d = rank_col.reshape(NPAD // 128, 128)
    sx = _sc_sortperm(rank2d, iota2d, sxu)
    z2 = _tc_conv(sx, w1k, conv1_b, w2k, conv2_b)
    h2 = _sc_pgather(z2, rank1d)
    pa = _tc_parta(h1, la, lin_b)
    return _tc_final(pa, h2, lb)


# speedup vs baseline: 22.0767x; 1.0039x over previous
import functools

import jax
import jax.numpy as jnp
from jax import lax
from jax.experimental import pallas as pl
from jax.experimental.pallas import tpu as pltpu
from jax.experimental.pallas import tpu_sc as plsc

N = 10000
E = 320000
D = 128
K = 5
NPAD = 10240
EP = 327680
ER = EP // 128
EPAD = EP - E
HALF = 64
EPT = EP // 16
RPT = EPT // 128
HRPT = ER // 32
GB = 256
NB = EPT // GB
NRT = NPAD // 16

_MESH = plsc.VectorSubcoreMesh(core_axis_name="c", subcore_axis_name="s",
                               num_cores=2, num_subcores=16)


@functools.partial(
    pl.kernel,
    out_type=jax.ShapeDtypeStruct((2 * NPAD,), jnp.float32),
    mesh=_MESH,
    scratch_types=[
        pltpu.VMEM((HRPT, 128), jnp.int32),
        pltpu.VMEM((128,), jnp.float32),
        pltpu.VMEM_SHARED((NPAD,), jnp.float32),
    ],
)
def _sc_hist(dst2d, init_half, out, idx_v, ones_v, hist_sp):
    c = lax.axis_index("c")
    s = lax.axis_index("s")

    @pl.when(s == 0)
    def _():
        pltpu.sync_copy(init_half, hist_sp)

    for i in range(8):
        ones_v[pl.ds(i * 16, 16)] = jnp.ones((16,), jnp.float32)

    pltpu.sync_copy(dst2d.at[pl.ds((c * 16 + s) * HRPT, HRPT)], idx_v)
    plsc.subcore_barrier()

    def body(j, carry):
        pltpu.sync_copy(ones_v, hist_sp.at[idx_v.at[j]], add=True)
        return carry

    lax.fori_loop(0, HRPT, body, 0)
    plsc.subcore_barrier()

    @pl.when(s == 0)
    def _():
        pltpu.sync_copy(hist_sp, out.at[pl.ds(c * NPAD, NPAD)])


@functools.partial(
    pl.kernel,
    out_type=jax.ShapeDtypeStruct((2 * NPAD, HALF), jnp.float32),
    mesh=_MESH,
    scratch_types=[
        pltpu.VMEM((EPT,), jnp.int32),
        pltpu.VMEM((RPT, 128), jnp.int32),
        pltpu.VMEM((GB, HALF), jnp.float32),
        pltpu.VMEM((GB, HALF), jnp.float32),
        pltpu.VMEM_SHARED((NPAD, HALF), jnp.float32),
        pltpu.SemaphoreType.DMA,
        pltpu.SemaphoreType.DMA,
        pltpu.SemaphoreType.DMA,
    ],
    compiler_params=pltpu.CompilerParams(use_tc_tiling_on_sc=False),
)
def _sc_rowscat(y2, src2, dst2d, out, src_v, dst_v, rows_a, rows_b, acc_sp,
                sem_a, sem_b, sem_s):
    c = lax.axis_index("c")
    s = lax.axis_index("s")

    pltpu.sync_copy(y2.at[pl.ds(c * NPAD + s * NRT, NRT)],
                    acc_sp.at[pl.ds(s * NRT, NRT)])
    pltpu.sync_copy(src2.at[pl.ds(c * EP + s * EPT, EPT)], src_v)
    pltpu.sync_copy(dst2d.at[pl.ds(s * RPT, RPT)], dst_v)
    plsc.subcore_barrier()

    def _issue(b, buf, sem):
        pltpu.async_copy(y2.at[src_v.at[pl.ds(b * GB, GB)]], buf, sem)

    def _wait(b, buf, sem):
        pltpu.make_async_copy(y2.at[src_v.at[pl.ds(b * GB, GB)]], buf,
                              sem).wait()

    def _scat(b, buf):
        for j in range(GB // 128):
            pltpu.async_copy(buf.at[pl.ds(j * 128, 128)],
                             acc_sp.at[dst_v.at[b * (GB // 128) + j]],
                             sem_s, add=True)
        for j in range(GB // 128):
            pltpu.make_async_copy(buf.at[pl.ds(j * 128, 128)],
                                  acc_sp.at[dst_v.at[b * (GB // 128) + j]],
                                  sem_s).wait()

    _issue(0, rows_a, sem_a)

    def body(b2, carry):
        b0 = 2 * b2
        _issue(b0 + 1, rows_b, sem_b)
        _wait(b0, rows_a, sem_a)
        _scat(b0, rows_a)

        @pl.when(b0 + 2 < NB)
        def _():
            _issue(b0 + 2, rows_a, sem_a)

        _wait(b0 + 1, rows_b, sem_b)
        _scat(b0 + 1, rows_b)
        return carry

    lax.fori_loop(0, NB // 2, body, 0)
    plsc.subcore_barrier()
    pltpu.sync_copy(acc_sp.at[pl.ds(s * NRT, NRT)],
                    out.at[pl.ds(c * NPAD + s * NRT, NRT)])


@functools.partial(
    pl.kernel,
    out_type=jax.ShapeDtypeStruct((NPAD, D), jnp.float32),
    mesh=_MESH,
    scratch_types=[
        pltpu.VMEM((8, 128), jnp.int32),
        pltpu.VMEM((8, 128), jnp.int32),
        pltpu.VMEM((NRT,), jnp.int32),
        pltpu.VMEM((NRT, D), jnp.float32),
        pltpu.VMEM_SHARED((NPAD,), jnp.int32),
        pltpu.SemaphoreType.DMA,
    ],
)
def _sc_sortperm(rank2d, iota2d, sxu, out, rank_v, vals_v, idx_v, rows_v,
                 order_sp, sem):
    c = lax.axis_index("c")
    s = lax.axis_index("s")

    @pl.when(c == 0)
    def _():
        @pl.when(s < 10)
        def _():
            pltpu.sync_copy(rank2d.at[pl.ds(s * 8, 8)], rank_v)
            pltpu.sync_copy(iota2d.at[pl.ds(s * 8, 8)], vals_v)
            for j in range(8):
                pltpu.sync_copy(vals_v.at[j], order_sp.at[rank_v.at[j]])

        plsc.subcore_barrier()
        base = s * NRT
        pltpu.sync_copy(order_sp.at[pl.ds(base, NRT)], idx_v)
        pltpu.async_copy(sxu.at[idx_v], rows_v, sem).wait()
        pltpu.sync_copy(rows_v, out.at[pl.ds(base, NRT)])


@functools.partial(
    pl.kernel,
    out_type=jax.ShapeDtypeStruct((NPAD, D), jnp.float32),
    mesh=_MESH,
    scratch_types=[
        pltpu.VMEM((NRT,), jnp.int32),
        pltpu.VMEM((NRT, D), jnp.float32),
        pltpu.SemaphoreType.DMA,
    ],
)
def _sc_pgather(table, idx, out, idx_v, rows_v, sem):
    c = lax.axis_index("c")
    s = lax.axis_index("s")

    @pl.when(c == 0)
    def _():
        base = s * NRT
        pltpu.sync_copy(idx.at[pl.ds(base, NRT)], idx_v)
        pltpu.async_copy(table.at[idx_v], rows_v, sem).wait()
        pltpu.sync_copy(rows_v, out.at[pl.ds(base, NRT)])


def _tc_y0_body(x_ref, h_ref, w_ref, y_ref):
    hh = h_ref[...]
    dinv = lax.rsqrt(hh[:, 0] + hh[:, 1])[:, None]
    y = jnp.dot(x_ref[...], w_ref[...],
                preferred_element_type=jnp.float32) * dinv
    y_ref[0] = y[:, :HALF]
    y_ref[1] = y[:, HALF:]


def _tc_mid_body(s_ref, h_ref, b_ref, w_ref, y_ref):
    hh = h_ref[...]
    dinv = lax.rsqrt(hh[:, 0] + hh[:, 1])[:, None]
    sfull = jnp.concatenate([s_ref[0], s_ref[1]], axis=1)
    hrelu = jnp.maximum(dinv * sfull + b_ref[...][None, :], 0.0)
    y = jnp.dot(hrelu, w_ref[...],
                preferred_element_type=jnp.float32) * dinv
    y_ref[0] = y[:, :HALF]
    y_ref[1] = y[:, HALF:]


def _tc_head_body(s_ref, h_ref, b_ref, p_ref, pb_ref,
                  h1_ref, kc_ref, sxu_ref):
    i = pl.program_id(0)
    B = 2048
    hh = h_ref[...]
    dinv = lax.rsqrt(hh[:, 0] + hh[:, 1])[:, None]
    sfull = jnp.concatenate([s_ref[0], s_ref[1]], axis=1)
    h1 = dinv * sfull + b_ref[...][None, :]
    g = jnp.sum(h1 * p_ref[...][None, :], axis=1, keepdims=True) + pb_ref[0, 0]
    rows = i * B + lax.broadcasted_iota(jnp.int32, (B, 1), 0)
    valid = rows < N
    bits = lax.bitcast_convert_type(g, jnp.int32)
    key = bits ^ (lax.shift_right_arithmetic(bits, 31) & jnp.int32(0x7FFFFFFF))
    h1_ref[...] = h1
    kc_ref[...] = jnp.where(valid, key, jnp.int32(0x7FFFFFFF))
    sxu_ref[...] = jnp.where(valid, g * h1, 0.0)


_RBI, _RBJ = 2048, 2048


def _tc_rank_body(kc_ref, kr_ref, o_ref):
    i = pl.program_id(0)
    j = pl.program_id(1)
    BI, BJ = _RBI, _RBJ
    ki = kc_ref[...]
    kj = kr_ref[...]

    @pl.when(j == 0)
    def _():
        o_ref[...] = jnp.zeros((BI, 1), jnp.int32)

    @pl.when(j < i)
    def _():
        o_ref[...] += jnp.sum((kj <= ki).astype(jnp.int32), axis=1,
                              keepdims=True)

    @pl.when(j > i)
    def _():
        o_ref[...] += jnp.sum((kj < ki).astype(jnp.int32), axis=1,
                              keepdims=True)

    @pl.when(j == i)
    def _():
        ii = i * BI + lax.broadcasted_iota(jnp.int32, (BI, 1), 0)
        jj = j * BJ + lax.broadcasted_iota(jnp.int32, (1, BJ), 1)
        cnt = ((kj < ki) | ((kj == ki) & (jj < ii))).astype(jnp.int32)
        o_ref[...] += jnp.sum(cnt, axis=1, keepdims=True)


def _tc_conv_body(prev_ref, cur_ref, next_ref, w1_ref, b1_ref,
                  w2_ref, b2_ref, o_ref):
    i = pl.program_id(0)
    B = 2048
    ext = jnp.concatenate(
        [prev_ref[...][B - 4:], cur_ref[...], next_ref[...][:4]], axis=0)
    r = i * B - 4 + lax.broadcasted_iota(jnp.int32, (B + 8, 1), 0)
    ext = jnp.where((r >= 0) & (r < N), ext, 0.0)
    z = jnp.zeros((B + 4, D), jnp.float32)
    for k in range(K):
        z = z + jnp.dot(ext[k:k + B + 4], w1_ref[k],
                        preferred_element_type=jnp.float32)
    z = jnp.maximum(z + b1_ref[...][None, :], 0.0)
    zr = i * B - 2 + lax.broadcasted_iota(jnp.int32, (B + 4, 1), 0)
    z = jnp.where((zr >= 0) & (zr < N), z, 0.0)
    out = jnp.zeros((B, D), jnp.float32)
    for k in range(K):
        out = out + jnp.dot(z[k:k + B], w2_ref[k],
                            preferred_element_type=jnp.float32)
    o_ref[...] = out + b2_ref[...][None, :]


def _tc_final_body(h1_ref, h2_ref, la_ref, lb_ref, b_ref, o_ref):
    o_ref[...] = (
        jnp.dot(h1_ref[...], la_ref[...], preferred_element_type=jnp.float32)
        + jnp.dot(h2_ref[...], lb_ref[...], preferred_element_type=jnp.float32)
        + b_ref[...][None, :])


def _full(shape):
    zeros = (0,) * len(shape)
    return pl.BlockSpec(shape, lambda *_: zeros)


def _tc_y0(x, hists, w0):
    B = 2048
    return pl.pallas_call(
        _tc_y0_body,
        grid=(NPAD // B,),
        in_specs=[pl.BlockSpec((B, D), lambda i: (i, 0)),
                  pl.BlockSpec((B, 2), lambda i: (i, 0)),
                  _full((D, D))],
        out_specs=pl.BlockSpec((2, B, HALF), lambda i: (0, i, 0)),
        out_shape=jax.ShapeDtypeStruct((2, NPAD, HALF), jnp.float32),
    )(x, hists, w0)


def _tc_mid(s0, hists, b0, w1):
    B = 2048
    return pl.pallas_call(
        _tc_mid_body,
        grid=(NPAD // B,),
        in_specs=[pl.BlockSpec((2, B, HALF), lambda i: (0, i, 0)),
                  pl.BlockSpec((B, 2), lambda i: (i, 0)),
                  _full((D,)), _full((D, D))],
        out_specs=pl.BlockSpec((2, B, HALF), lambda i: (0, i, 0)),
        out_shape=jax.ShapeDtypeStruct((2, NPAD, HALF), jnp.float32),
    )(s0, hists, b0, w1)


def _tc_head(s1, hists, b1, proj_row, pb):
    B = 2048
    return pl.pallas_call(
        _tc_head_body,
        grid=(NPAD // B,),
        in_specs=[pl.BlockSpec((2, B, HALF), lambda i: (0, i, 0)),
                  pl.BlockSpec((B, 2), lambda i: (i, 0)),
                  _full((D,)), _full((D,)), _full((1, 1))],
        out_specs=[pl.BlockSpec((B, D), lambda i: (i, 0)),
                   pl.BlockSpec((B, 1), lambda i: (i, 0)),
                   pl.BlockSpec((B, D), lambda i: (i, 0))],
        out_shape=[jax.ShapeDtypeStruct((N, D), jnp.float32),
                   jax.ShapeDtypeStruct((NPAD, 1), jnp.int32),
                   jax.ShapeDtypeStruct((NPAD, D), jnp.float32)],
    )(s1, hists, b1, proj_row, pb)


def _tc_rank(keys_col, keys_row):
    BI, BJ = _RBI, _RBJ
    return pl.pallas_call(
        _tc_rank_body,
        grid=(NPAD // BI, NPAD // BJ),
        in_specs=[pl.BlockSpec((BI, 1), lambda i, j: (i, 0)),
                  pl.BlockSpec((1, BJ), lambda i, j: (0, j))],
        out_specs=pl.BlockSpec((BI, 1), lambda i, j: (i, 0)),
        out_shape=jax.ShapeDtypeStruct((NPAD, 1), jnp.int32),
    )(keys_col, keys_row)


def _tc_conv(sx, w1k, b1, w2k, b2):
    B = 2048
    nb = NPAD // B
    return pl.pallas_call(
        _tc_conv_body,
        grid=(nb,),
        in_specs=[
            pl.BlockSpec((B, D), lambda i: (jnp.maximum(i - 1, 0), 0)),
            pl.BlockSpec((B, D), lambda i: (i, 0)),
            pl.BlockSpec((B, D), lambda i: (jnp.minimum(i + 1, nb - 1), 0)),
            _full((K, D, D)), _full((D,)), _full((K, D, D)), _full((D,)),
        ],
        out_specs=pl.BlockSpec((B, D), lambda i: (i, 0)),
        out_shape=jax.ShapeDtypeStruct((NPAD, D), jnp.float32),
    )(sx, sx, sx, w1k, b1, w2k, b2)


def _tc_final(h1, h2, la, lb, lbias):
    B = 2000
    return pl.pallas_call(
        _tc_final_body,
        grid=(N // B,),
        in_specs=[pl.BlockSpec((B, D), lambda i: (i, 0)),
                  pl.BlockSpec((B, D), lambda i: (i, 0)),
                  _full((D, D)), _full((D, D)), _full((D,))],
        out_specs=pl.BlockSpec((B, D), lambda i: (i, 0)),
        out_shape=jax.ShapeDtypeStruct((N, D), jnp.float32),
    )(h1, h2, la, lb, lbias)


def kernel(x, edge_index, gcn0_w, gcn0_b, gcn1_w, gcn1_b, proj_w, proj_b,
           conv1_w, conv1_b, conv2_w, conv2_b, lin_w, lin_b):
    import numpy as _np
    pad_k = _np.arange(EPAD, dtype=_np.int32) % 64
    src1d = jnp.concatenate([edge_index[0], jnp.asarray(pad_k)])
    src2 = jnp.concatenate([src1d, src1d + NPAD])
    dst2d = jnp.concatenate(
        [edge_index[1], jnp.asarray(N + pad_k)]).reshape(ER, 128)
    init_half = jnp.full((NPAD,), 0.5, jnp.float32)
    iota2d = jnp.asarray(
        _np.arange(NPAD, dtype=_np.int32).reshape(NPAD // 128, 128))
    w1k = jnp.transpose(conv1_w, (2, 1, 0))
    w2k = jnp.transpose(conv2_w, (2, 1, 0))
    la = lin_w[:, :D].T
    lb = lin_w[:, D:].T

    hflat = _sc_hist(dst2d, init_half)
    hists = hflat.reshape(2, NPAD).T
    y0 = _tc_y0(x, hists, gcn0_w)
    s0 = _sc_rowscat(y0.reshape(2 * NPAD, HALF),
                     src2, dst2d).reshape(2, NPAD, HALF)
    y1 = _tc_mid(s0, hists, gcn0_b, gcn1_w)
    s1 = _sc_rowscat(y1.reshape(2 * NPAD, HALF),
                     src2, dst2d).reshape(2, NPAD, HALF)
    h1, keys_col, sxu = _tc_head(s1, hists, gcn1_b,
                                 proj_w[0], proj_b.reshape(1, 1))
    keys_row = keys_col.reshape(1, NPAD)
    rank_col = _tc_rank(keys_col, keys_row)
    rank1d = rank_col.reshape(NPAD)
    rank2d = rank_col.reshape(NPAD // 128, 128)
    sx = _sc_sortperm(rank2d, iota2d, sxu)
    z2 = _tc_conv(sx, w1k, conv1_b, w2k, conv2_b)
    h2 = _sc_pgather(z2, rank1d)
    return _tc_final(h1, h2, la, lb, lin_b)
